# baseline (speedup 1.0000x reference)
import functools

import jax
import jax.numpy as jnp
from jax import lax
from jax.experimental import pallas as pl
from jax.experimental.pallas import tpu as pltpu
from jax.experimental.pallas import tpu_sc as plsc

_N_COARSE = 50000
_N_FINE = 100000
_D = 128
_CH = 128
_NW = 32
_NFULL = _N_FINE // _CH
_TAIL = _N_FINE - _NFULL * _CH
_NCHUNKS = _NFULL + 1
_NTRIPLES = 8
_L = 16
_FL = 8

_mesh = plsc.VectorSubcoreMesh(core_axis_name="c", subcore_axis_name="s")


def _zero_shadow_rows(save_ref, rows_ref, z_hbm, n):
    def body(i, carry):
        orig = save_ref[pl.ds(i, _L)][0]

        @pl.when(orig == _N_COARSE)
        def _():
            pltpu.sync_copy(z_hbm.at[pl.ds(0, 1)], rows_ref.at[pl.ds(i, 1)])
        return carry

    lax.fori_loop(0, n, body, 0)


@functools.partial(
    pl.kernel,
    out_type=jax.ShapeDtypeStruct((_N_FINE, _D), jnp.float32),
    mesh=_mesh,
    scratch_types=[
        pltpu.VMEM((3, _CH), jnp.int32),
        pltpu.VMEM((_L,), jnp.int32),
        pltpu.VMEM((_L,), jnp.int32),
        pltpu.VMEM((_L,), jnp.int32),
        pltpu.VMEM((3, _CH, _D), jnp.float32),
        pltpu.VMEM((_CH + _L,), jnp.int32),
        pltpu.VMEM((_TAIL,), jnp.int32),
        pltpu.VMEM((_L,), jnp.int32),
        pltpu.VMEM((_TAIL, _D), jnp.float32),
        pltpu.SemaphoreType.DMA,
        pltpu.SemaphoreType.DMA,
        pltpu.SemaphoreType.DMA,
        pltpu.SemaphoreType.DMA,
        pltpu.SemaphoreType.DMA,
        pltpu.SemaphoreType.DMA,
        pltpu.SemaphoreType.DMA,
        pltpu.SemaphoreType.DMA,
        pltpu.SemaphoreType.DMA,
        pltpu.SemaphoreType.DMA,
    ],
)
def _sc_gather(x_hbm, idxc_hbm, raw_hbm, flg_hbm, z_hbm, out_hbm,
               idx_v, flg0, flg1, flg2, rows_v, idx_s, idx_t, flg_t, rows_t,
               si0, si1, si2, sg0, sg1, sg2, sw0, sw1, sw2, st):
    flg_v = (flg0, flg1, flg2)
    wid = lax.axis_index("s") * 2 + lax.axis_index("c")
    nc = jnp.where(wid <= 12, 25, 24)

    sem_i = (si0, si1, si2)
    sem_g = (sg0, sg1, sg2)
    sem_w = (sw0, sw1, sw2)

    def issue_idx(chunk, b, sem):
        base = chunk * _CH
        pltpu.async_copy(idxc_hbm.at[pl.ds(base, _CH)], idx_v.at[b], sem)
        pltpu.async_copy(flg_hbm.at[pl.ds(chunk * _FL, _FL)],
                         flg_v[b].at[pl.ds(0, _FL)], sem)

    def drain_idx(b):
        pltpu.make_async_copy(idxc_hbm.at[pl.ds(0, _CH)], idx_v.at[b],
                              sem_i[b]).wait()
        pltpu.make_async_copy(flg_hbm.at[pl.ds(0, _FL)],
                              flg_v[b].at[pl.ds(0, _FL)], sem_i[b]).wait()

    def finish_chunk(c, b):
        base = (wid + c * _NW) * _CH
        my_rows = rows_v.at[b]
        pltpu.make_async_copy(x_hbm.at[idx_v.at[b]], my_rows,
                              sem_g[b]).wait()

        @pl.when(flg_v[b][pl.ds(0, _L)][0] != 0)
        def _():
            pltpu.sync_copy(raw_hbm.at[pl.ds(base, _CH)],
                            idx_s.at[pl.ds(0, _CH)])
            _zero_shadow_rows(idx_s, my_rows, z_hbm, _CH)

        pltpu.async_copy(my_rows, out_hbm.at[pl.ds(base, _CH)], sem_w[b])

        @pl.when(c + 3 < nc)
        def _():
            issue_idx(wid + (c + 3) * _NW, b, sem_i[b])

    def chunk_step(c, b):
        my_idx = idx_v.at[b]
        my_rows = rows_v.at[b]

        @pl.when(c == 0)
        def _():
            for s in range(3):
                issue_idx(wid + s * _NW, s, sem_i[s])

        drain_idx(b)

        @pl.when(c >= 3)
        def _():
            pltpu.make_async_copy(my_rows, out_hbm.at[pl.ds(0, _CH)],
                                  sem_w[b]).wait()

        pltpu.async_copy(x_hbm.at[my_idx], my_rows, sem_g[b])

        @pl.when(c >= 1)
        def _():
            finish_chunk(c - 1, (b + 2) % 3)

    def triple_body(p, carry):
        chunk_step(3 * p, 0)
        chunk_step(3 * p + 1, 1)
        chunk_step(3 * p + 2, 2)
        return carry

    lax.fori_loop(0, _NTRIPLES, triple_body, 0)

    @pl.when(nc == 25)
    def _():
        chunk_step(jnp.int32(24), 0)

    @pl.when(nc == 25)
    def _():
        finish_chunk(jnp.int32(24), 0)

    @pl.when(nc == 24)
    def _():
        finish_chunk(jnp.int32(23), 2)

    for s, sw in enumerate((sw0, sw1, sw2)):
        pltpu.make_async_copy(rows_v.at[s], out_hbm.at[pl.ds(0, _CH)],
                              sw).wait()

    @pl.when(wid == _NW - 1)
    def _():
        tbase = _NFULL * _CH
        pltpu.sync_copy(idxc_hbm.at[pl.ds(tbase, _TAIL)], idx_t)
        pltpu.sync_copy(flg_hbm.at[pl.ds(_NFULL * _FL, _FL)],
                        flg_t.at[pl.ds(0, _FL)])
        pltpu.async_copy(x_hbm.at[idx_t], rows_t, st).wait()

        @pl.when(flg_t[pl.ds(0, _L)][0] != 0)
        def _():
            pltpu.sync_copy(raw_hbm.at[pl.ds(tbase, _TAIL)],
                            idx_s.at[pl.ds(0, _TAIL)])
            _zero_shadow_rows(idx_s, rows_t, z_hbm, _TAIL)

        pltpu.sync_copy(rows_t, out_hbm.at[pl.ds(tbase, _TAIL)])


def kernel(x, upsamples):
    raw = upsamples[:, 0]
    idxc = jnp.minimum(raw, _N_COARSE - 1)
    pad = jnp.zeros((_NCHUNKS * _CH - _N_FINE,), raw.dtype)
    shadow = (jnp.concatenate([raw, pad]) >= _N_COARSE).astype(jnp.int32)
    flags = jnp.max(shadow.reshape(_NCHUNKS, _CH), axis=1)
    flags8 = jnp.broadcast_to(flags[:, None], (_NCHUNKS, _FL)).reshape(-1)
    z = jnp.zeros((1, _D), x.dtype)
    return _sc_gather(x, idxc, raw, flags8, z)
